# baseline (speedup 1.0000x reference)
import functools

import jax
import jax.numpy as jnp
from jax import lax
from jax.experimental import pallas as pl
from jax.experimental.pallas import tpu as pltpu
from jax.experimental.pallas import tpu_sc as plsc

M = 4096
N = 4096
NC = 2
NS = 16
NW = NC * NS
M_SC = 512
ROWS_W = M_SC // NW
VARS_W = N // NW
PK = 2 * ROWS_W + 4 * VARS_W

BM = 256
M_TC = M - M_SC
NB = M_TC // BM

_mesh = plsc.VectorSubcoreMesh(core_axis_name="c", subcore_axis_name="s")


@functools.partial(
    pl.kernel,
    out_type=jax.ShapeDtypeStruct((NW, 3, 16), jnp.float32),
    mesh=_mesh,
    scratch_types=(
        pltpu.VMEM((ROWS_W, N), jnp.float32),
        pltpu.VMEM((N,), jnp.float32),
        pltpu.VMEM((PK,), jnp.float32),
        pltpu.VMEM((3, 16), jnp.float32),
        pltpu.SemaphoreType.DMA,
    ),
)
def _sc_partials(a_hbm, x_hbm, pk_hbm, out_hbm, a_v, x_v, pk_v, o_v, sem):
    wid = lax.axis_index("s") * NC + lax.axis_index("c")
    row0 = wid * ROWS_W
    var0 = wid * VARS_W

    a_copy = pltpu.async_copy(a_hbm.at[pl.ds(row0, ROWS_W)], a_v, sem)
    pltpu.sync_copy(x_hbm, x_v)
    pltpu.sync_copy(pk_hbm.at[wid], pk_v)

    zero = jnp.zeros((16,), jnp.float32)
    lane = lax.iota(jnp.int32, 16)
    OFF_IY = ROWS_W
    OFF_L = 2 * ROWS_W
    OFF_U = OFF_L + VARS_W
    OFF_IL = OFF_U + VARS_W
    OFF_IU = OFF_IL + VARS_W

    m_stk = zero
    for t in range(VARS_W // 16):
        xv = x_v[pl.ds(var0 + t * 16, 16)]
        lv = pk_v[pl.ds(OFF_L + t * 16, 16)]
        uv = pk_v[pl.ds(OFF_U + t * 16, 16)]
        ilv = pk_v[pl.ds(OFF_IL + t * 16, 16)]
        iuv = pk_v[pl.ds(OFF_IU + t * 16, 16)]
        v = jnp.maximum(lv - xv, 0.0) * ilv + jnp.maximum(xv - uv, 0.0) * iuv
        m_stk = jnp.maximum(m_stk, jnp.abs(v))

    a_copy.wait()

    sums = zero
    for half in range(2):
        @plsc.parallel_loop(0, N, step=16, unroll=4, carry=(zero,) * 8)
        def accs(j, accs, half=half):
            xv = x_v[pl.ds(j, 16)]
            return tuple(
                accs[r] + a_v[half * 8 + r, pl.ds(j, 16)] * xv
                for r in range(8)
            )

        for r in range(8):
            v = accs[r]
            for sh in (8, 4, 2, 1):
                idx = lax.bitwise_xor(lane, sh)
                v = v + v.at[idx].get(mode="promise_in_bounds")
            sums = jnp.where(lane == (half * 8 + r), v, sums)

    bvec = pk_v[pl.ds(0, 16)]
    iyv = pk_v[pl.ds(OFF_IY, 16)]
    cons = bvec - sums
    cons = cons + jnp.maximum(-cons, 0.0) * iyv
    m_stk = jnp.maximum(m_stk, jnp.abs(cons))

    o_v[0, :] = m_stk
    o_v[1, :] = jnp.abs(sums)
    o_v[2, :] = jnp.abs(bvec)
    pltpu.sync_copy(o_v, out_hbm.at[wid])


def _tc_body(a_ref, xr_ref, b_ref, iy_ref, stk_ref, ax_ref, bmx_ref):
    ax = jnp.sum(a_ref[...] * xr_ref[...], axis=1)
    bv = b_ref[...]
    cons = bv - ax
    cons = cons + jnp.maximum(-cons, 0.0) * iy_ref[...]
    stk_ref[...] = jnp.full((1, 1, 128), jnp.max(jnp.abs(cons)), jnp.float32)
    ax_ref[...] = jnp.full((1, 1, 128), jnp.max(jnp.abs(ax)), jnp.float32)
    bmx_ref[...] = jnp.full((1, 1, 128), jnp.max(jnp.abs(bv)), jnp.float32)


_tc_partials = pl.pallas_call(
    _tc_body,
    grid=(NB,),
    in_specs=[
        pl.BlockSpec((BM, N), lambda i: (M_SC // BM + i, 0)),
        pl.BlockSpec((1, N), lambda i: (0, 0)),
        pl.BlockSpec((BM,), lambda i: (M_SC // BM + i,)),
        pl.BlockSpec((BM,), lambda i: (M_SC // BM + i,)),
    ],
    out_specs=[
        pl.BlockSpec((1, 1, 128), lambda i: (i, 0, 0)),
        pl.BlockSpec((1, 1, 128), lambda i: (i, 0, 0)),
        pl.BlockSpec((1, 1, 128), lambda i: (i, 0, 0)),
    ],
    out_shape=[
        jax.ShapeDtypeStruct((NB, 1, 128), jnp.float32),
        jax.ShapeDtypeStruct((NB, 1, 128), jnp.float32),
        jax.ShapeDtypeStruct((NB, 1, 128), jnp.float32),
    ],
)


def _combine_body(p_ref, s1_ref, a1_ref, b1_ref, o_ref):
    stk = jnp.maximum(jnp.max(p_ref[:, 0, :]), jnp.max(s1_ref[...]))
    axm = jnp.maximum(jnp.max(p_ref[:, 1, :]), jnp.max(a1_ref[...]))
    bmx = jnp.maximum(jnp.max(p_ref[:, 2, :]), jnp.max(b1_ref[...]))
    o_ref[...] = jnp.reshape(stk / (1.0 + jnp.maximum(axm, bmx)), (1, 1))


def kernel(A, b, c, x, Iy, il, iu, l, u):
    del c
    pack = jnp.concatenate(
        [
            b[:M_SC].reshape(NW, ROWS_W),
            Iy[:M_SC].reshape(NW, ROWS_W),
            l.reshape(NW, VARS_W),
            u.reshape(NW, VARS_W),
            il.reshape(NW, VARS_W),
            iu.reshape(NW, VARS_W),
        ],
        axis=1,
    )
    p = _sc_partials(A, x.reshape(N), pack)
    s1, a1, b1 = _tc_partials(A, x.reshape(1, N), b, Iy.reshape(M))
    out = pl.pallas_call(
        _combine_body,
        out_shape=jax.ShapeDtypeStruct((1, 1), jnp.float32),
    )(p, s1, a1, b1)
    return out[0, 0]


# speedup vs baseline: 1.6476x; 1.6476x over previous
import jax
import jax.numpy as jnp
from jax.experimental import pallas as pl
from jax.experimental.pallas import tpu as pltpu

M = 4096
N = 4096
BM = 256
NB = M // BM


def _body(a_ref, xr_ref, b_ref, iy_ref, l_ref, u_ref, il_ref, iu_ref,
          o_ref, acc_ref):
    i = pl.program_id(0)

    @pl.when(i == 0)
    def _init():
        xv = xr_ref[...]
        var = (jnp.maximum(l_ref[...] - xv, 0.0) * il_ref[...]
               + jnp.maximum(xv - u_ref[...], 0.0) * iu_ref[...])
        acc_ref[0:1, :] = jnp.full((1, 128), jnp.max(jnp.abs(var)), jnp.float32)
        acc_ref[1:2, :] = jnp.zeros((1, 128), jnp.float32)
        acc_ref[2:3, :] = jnp.zeros((1, 128), jnp.float32)

    ax = jnp.sum(a_ref[...] * xr_ref[...], axis=1)
    bv = b_ref[...]
    cons = bv - ax
    cons = cons + jnp.maximum(-cons, 0.0) * iy_ref[...]
    acc_ref[0:1, :] = jnp.maximum(
        acc_ref[0:1, :], jnp.full((1, 128), jnp.max(jnp.abs(cons)), jnp.float32))
    acc_ref[1:2, :] = jnp.maximum(
        acc_ref[1:2, :], jnp.full((1, 128), jnp.max(jnp.abs(ax)), jnp.float32))
    acc_ref[2:3, :] = jnp.maximum(
        acc_ref[2:3, :], jnp.full((1, 128), jnp.max(jnp.abs(bv)), jnp.float32))

    @pl.when(i == NB - 1)
    def _finish():
        stk = jnp.max(acc_ref[0:1, :])
        axm = jnp.max(acc_ref[1:2, :])
        bmx = jnp.max(acc_ref[2:3, :])
        o_ref[...] = jnp.reshape(stk / (1.0 + jnp.maximum(axm, bmx)), (1, 1))


_fused = pl.pallas_call(
    _body,
    grid=(NB,),
    in_specs=[
        pl.BlockSpec((BM, N), lambda i: (i, 0)),
        pl.BlockSpec((1, N), lambda i: (0, 0)),
        pl.BlockSpec((BM,), lambda i: (i,)),
        pl.BlockSpec((BM,), lambda i: (i,)),
        pl.BlockSpec((1, N), lambda i: (0, 0)),
        pl.BlockSpec((1, N), lambda i: (0, 0)),
        pl.BlockSpec((1, N), lambda i: (0, 0)),
        pl.BlockSpec((1, N), lambda i: (0, 0)),
    ],
    out_specs=pl.BlockSpec((1, 1), lambda i: (0, 0)),
    out_shape=jax.ShapeDtypeStruct((1, 1), jnp.float32),
    scratch_shapes=[pltpu.VMEM((3, 128), jnp.float32)],
)


def kernel(A, b, c, x, Iy, il, iu, l, u):
    del c
    out = _fused(A, x.reshape(1, N), b, Iy.reshape(M),
                 l.reshape(1, N), u.reshape(1, N),
                 il.reshape(1, N), iu.reshape(1, N))
    return out[0, 0]


# speedup vs baseline: 2.0100x; 1.2200x over previous
import jax
import jax.numpy as jnp
from jax.experimental import pallas as pl
from jax.experimental.pallas import tpu as pltpu

M = 4096
N = 4096
BM = 256
NB = M // BM // 2


def _body(a_ref, a2_ref, xr_ref, b_ref, b2_ref, iy_ref, iy2_ref,
          l_ref, u_ref, il_ref, iu_ref, o_ref, acc_ref):
    i = pl.program_id(0)

    @pl.when(i == 0)
    def _init():
        xv = xr_ref[...]
        var = (jnp.maximum(l_ref[...] - xv, 0.0) * il_ref[...]
               + jnp.maximum(xv - u_ref[...], 0.0) * iu_ref[...])
        acc_ref[0:1, :] = jnp.full((1, 128), jnp.max(jnp.abs(var)), jnp.float32)
        acc_ref[1:2, :] = jnp.zeros((1, 128), jnp.float32)
        acc_ref[2:3, :] = jnp.zeros((1, 128), jnp.float32)

    for ar, br, iyr in ((a_ref, b_ref, iy_ref), (a2_ref, b2_ref, iy2_ref)):
        ax = jnp.sum(ar[...] * xr_ref[...], axis=1)
        bv = br[...]
        cons = bv - ax
        cons = cons + jnp.maximum(-cons, 0.0) * iyr[...]
        acc_ref[0:1, :] = jnp.maximum(
            acc_ref[0:1, :], jnp.full((1, 128), jnp.max(jnp.abs(cons)), jnp.float32))
        acc_ref[1:2, :] = jnp.maximum(
            acc_ref[1:2, :], jnp.full((1, 128), jnp.max(jnp.abs(ax)), jnp.float32))
        acc_ref[2:3, :] = jnp.maximum(
            acc_ref[2:3, :], jnp.full((1, 128), jnp.max(jnp.abs(bv)), jnp.float32))

    @pl.when(i == NB - 1)
    def _finish():
        stk = jnp.max(acc_ref[0:1, :])
        axm = jnp.max(acc_ref[1:2, :])
        bmx = jnp.max(acc_ref[2:3, :])
        o_ref[...] = jnp.reshape(stk / (1.0 + jnp.maximum(axm, bmx)), (1, 1))


_fused = pl.pallas_call(
    _body,
    grid=(NB,),
    in_specs=[
        pl.BlockSpec((BM, N), lambda i: (i, 0)),
        pl.BlockSpec((BM, N), lambda i: (NB + i, 0)),
        pl.BlockSpec((1, N), lambda i: (0, 0)),
        pl.BlockSpec((BM,), lambda i: (i,)),
        pl.BlockSpec((BM,), lambda i: (NB + i,)),
        pl.BlockSpec((BM,), lambda i: (i,)),
        pl.BlockSpec((BM,), lambda i: (NB + i,)),
        pl.BlockSpec((1, N), lambda i: (0, 0)),
        pl.BlockSpec((1, N), lambda i: (0, 0)),
        pl.BlockSpec((1, N), lambda i: (0, 0)),
        pl.BlockSpec((1, N), lambda i: (0, 0)),
    ],
    out_specs=pl.BlockSpec((1, 1), lambda i: (0, 0)),
    out_shape=jax.ShapeDtypeStruct((1, 1), jnp.float32),
    scratch_shapes=[pltpu.VMEM((3, 128), jnp.float32)],
)


def kernel(A, b, c, x, Iy, il, iu, l, u):
    del c
    out = _fused(A, A, x.reshape(1, N), b, b, Iy.reshape(M), Iy.reshape(M),
                 l.reshape(1, N), u.reshape(1, N),
                 il.reshape(1, N), iu.reshape(1, N))
    return out[0, 0]
